# baseline (speedup 1.0000x reference)
import functools

import jax
import jax.numpy as jnp
from jax import lax
from jax.experimental import pallas as pl
from jax.experimental.pallas import tpu as pltpu
from jax.experimental.pallas import tpu_sc as plsc

B, S, D = 16, 4096, 256
NW = 32
ROWS_PER_W = B * S // NW
CHUNK = 64
NCHUNK = ROWS_PER_W // CHUNK

_mesh = plsc.VectorSubcoreMesh(core_axis_name="c", subcore_axis_name="s")


@functools.partial(
    pl.kernel,
    mesh=_mesh,
    out_type=jax.ShapeDtypeStruct((B * S, D), jnp.float32),
    scratch_types=[
        pltpu.VMEM((NCHUNK, CHUNK), jnp.int32),
    ] + [pltpu.VMEM((CHUNK, D), jnp.float32)] * 6
      + [pltpu.SemaphoreType.DMA] * 12,
)
def _zcurve_sc(x_hbm, idx_hbm, out_hbm, gidx_v, *bs):
    bufs, gsems, ssems = bs[0:6], bs[6:12], bs[12:18]
    wid = lax.axis_index("s") * 2 + lax.axis_index("c")
    out_base = wid * ROWS_PER_W

    pltpu.sync_copy(idx_hbm.at[wid], gidx_v)

    NBUF = 6
    g_copies = [None] * NBUF
    s_copies = [None] * NBUF
    for c in range(NCHUNK):
        p = c % NBUF
        if c >= NBUF:
            s_copies[p].wait()
        g_copies[p] = pltpu.async_copy(x_hbm.at[gidx_v.at[c]], bufs[p], gsems[p])
        if c >= 3:
            q = (c - 3) % NBUF
            g_copies[q].wait()
            s_copies[q] = pltpu.async_copy(
                bufs[q], out_hbm.at[pl.ds(out_base + (c - 3) * CHUNK, CHUNK)],
                ssems[q],
            )
    for c in range(max(0, NCHUNK - 3), NCHUNK):
        q = c % NBUF
        g_copies[q].wait()
        s_copies[q] = pltpu.async_copy(
            bufs[q], out_hbm.at[pl.ds(out_base + c * CHUNK, CHUNK)],
            ssems[q],
        )
    for c in range(max(0, NCHUNK - NBUF), NCHUNK):
        s_copies[c % NBUF].wait()


def kernel(x, forward_shuffle_idx):
    idx3 = forward_shuffle_idx.reshape(2, NCHUNK, CHUNK)
    gidx = idx3[None, :, :, :] + (jnp.arange(B, dtype=jnp.int32) * S)[
        :, None, None, None
    ]
    gidx = gidx.reshape(NW, NCHUNK, CHUNK)
    out = _zcurve_sc(x.reshape(B * S, D), gidx)
    return out.reshape(B, S, D)


# speedup vs baseline: 1.0026x; 1.0026x over previous
import functools

import jax
import jax.numpy as jnp
from jax import lax
from jax.experimental import pallas as pl
from jax.experimental.pallas import tpu as pltpu
from jax.experimental.pallas import tpu_sc as plsc

B, S, D = 16, 4096, 256
NW = 32
ROWS_PER_W = B * S // NW
CHUNK = 128
NCHUNK = ROWS_PER_W // CHUNK

_mesh = plsc.VectorSubcoreMesh(core_axis_name="c", subcore_axis_name="s")


@functools.partial(
    pl.kernel,
    mesh=_mesh,
    out_type=jax.ShapeDtypeStruct((B * S, D), jnp.float32),
    scratch_types=[
        pltpu.VMEM((NCHUNK, CHUNK), jnp.int32),
        pltpu.VMEM((CHUNK, D), jnp.float32),
        pltpu.VMEM((CHUNK, D), jnp.float32),
        pltpu.VMEM((CHUNK, D), jnp.float32),
        pltpu.SemaphoreType.DMA,
        pltpu.SemaphoreType.DMA,
        pltpu.SemaphoreType.DMA,
        pltpu.SemaphoreType.DMA,
        pltpu.SemaphoreType.DMA,
        pltpu.SemaphoreType.DMA,
    ],
)
def _zcurve_sc(x_hbm, idx_hbm, out_hbm, gidx_v,
               rows0_v, rows1_v, rows2_v,
               gsem0, gsem1, gsem2, ssem0, ssem1, ssem2):
    wid = lax.axis_index("s") * 2 + lax.axis_index("c")
    out_base = wid * ROWS_PER_W

    pltpu.sync_copy(idx_hbm.at[wid], gidx_v)

    NBUF = 3
    bufs = (rows0_v, rows1_v, rows2_v)
    gsems = (gsem0, gsem1, gsem2)
    ssems = (ssem0, ssem1, ssem2)
    g_copies = [None] * NBUF
    s_copies = [None] * NBUF
    for c in range(NCHUNK):
        p = c % NBUF
        if c >= NBUF:
            s_copies[p].wait()
        g_copies[p] = pltpu.async_copy(x_hbm.at[gidx_v.at[c]], bufs[p], gsems[p])
        if c >= 2:
            q = (c - 2) % NBUF
            g_copies[q].wait()
            s_copies[q] = pltpu.async_copy(
                bufs[q], out_hbm.at[pl.ds(out_base + (c - 2) * CHUNK, CHUNK)],
                ssems[q],
            )
    for c in range(max(0, NCHUNK - 2), NCHUNK):
        q = c % NBUF
        g_copies[q].wait()
        s_copies[q] = pltpu.async_copy(
            bufs[q], out_hbm.at[pl.ds(out_base + c * CHUNK, CHUNK)],
            ssems[q],
        )
    for c in range(max(0, NCHUNK - NBUF), NCHUNK):
        s_copies[c % NBUF].wait()


def kernel(x, forward_shuffle_idx):
    idx3 = forward_shuffle_idx.reshape(2, NCHUNK, CHUNK)
    gidx = idx3[None, :, :, :] + (jnp.arange(B, dtype=jnp.int32) * S)[
        :, None, None, None
    ]
    gidx = gidx.reshape(NW, NCHUNK, CHUNK)
    out = _zcurve_sc(x.reshape(B * S, D), gidx)
    return out.reshape(B, S, D)
